# baseline (speedup 1.0000x reference)
import jax
import jax.numpy as jnp
from jax import lax
from jax.experimental import pallas as pl
from jax.experimental.pallas import tpu as pltpu

NUM_POSITIONS = 8
CH = 16
ROW = NUM_POSITIONS * CH

BB = 16384


def _body(s_ref, x_ref, o_ref):
    blk = x_ref[...]
    bb = blk.shape[1]
    a = jnp.zeros((1, bb), jnp.float32)
    b = jnp.zeros((1, bb), jnp.float32)
    for q in range(NUM_POSITIONS):
        a = a + s_ref[q] * blk[CH * q : CH * q + 1, :]
        b = b + s_ref[q] * blk[CH * q + 1 : CH * q + 2, :]
    rid = lax.broadcasted_iota(jnp.int32, (ROW, 1), 0) % CH
    out = jnp.where(rid == 12, a, blk)
    out = jnp.where(rid == 13, b, out)
    o_ref[...] = out


def kernel(x, scales):
    B = x.shape[0]
    bb = BB
    while B % bb:
        bb //= 2
    xt = jnp.transpose(x, (1, 2, 0)).reshape(ROW, B)
    grid = (B // bb,)
    yt = pl.pallas_call(
        _body,
        grid=grid,
        in_specs=[
            pl.BlockSpec(memory_space=pltpu.SMEM),
            pl.BlockSpec((ROW, bb), lambda i: (0, i)),
        ],
        out_specs=pl.BlockSpec((ROW, bb), lambda i: (0, i)),
        out_shape=jax.ShapeDtypeStruct((ROW, B), x.dtype),
    )(scales, xt)
    return jnp.transpose(yt.reshape(NUM_POSITIONS, CH, B), (2, 0, 1))
